# baseline (speedup 1.0000x reference)
import functools

import jax
import jax.numpy as jnp
from jax import lax
from jax.experimental import pallas as pl
from jax.experimental.pallas import tpu as pltpu
from jax.experimental.pallas import tpu_sc as plsc

_L = 3
_NC, _NS, _LANES = 2, 16, 16
_NW = _NC * _NS


def _sc_msg_body(h_hbm, src_hbm, dst_hbm, ea_hbm, out_hbm,
                 src_v, dst_v, ea_v, rows_v, zero_v, out_sh, sem):
    c = lax.axis_index("c")
    s = lax.axis_index("s")
    wid = s * _NC + c
    bn = out_sh.shape[0]
    epw = ea_v.shape[0]
    nchunk = epw // 128
    rows_per_sub = bn // _NS

    for i in range(rows_per_sub):
        for j in range(8):
            zero_v[i, pl.ds(j * _LANES, _LANES)] = jnp.zeros(
                (_LANES,), jnp.float32)
    pltpu.sync_copy(zero_v, out_sh.at[pl.ds(s * rows_per_sub, rows_per_sub)])

    pltpu.sync_copy(src_hbm.at[pl.ds(wid * nchunk, nchunk)], src_v)
    pltpu.sync_copy(dst_hbm.at[pl.ds(wid * nchunk, nchunk)], dst_v)
    pltpu.sync_copy(ea_hbm.at[pl.ds(wid * epw, epw)], ea_v)

    for ch in range(nchunk):
        pltpu.async_copy(h_hbm.at[src_v.at[ch]],
                         rows_v.at[pl.ds(ch * 128, 128)], sem).wait()

    def _mul(e, carry):
        for j in range(8):
            sl = pl.ds(j * _LANES, _LANES)
            rows_v[e, sl] = rows_v[e, sl] * ea_v[e, sl]
        return carry
    lax.fori_loop(0, epw, _mul, 0)

    plsc.subcore_barrier()
    for ch in range(nchunk):
        pltpu.sync_copy(rows_v.at[pl.ds(ch * 128, 128)],
                        out_sh.at[dst_v.at[ch]], add=True)
    plsc.subcore_barrier()

    base = c * bn + s * rows_per_sub
    pltpu.sync_copy(out_sh.at[pl.ds(s * rows_per_sub, rows_per_sub)],
                    out_hbm.at[pl.ds(base, rows_per_sub)])


def _make_sc_msg(bn, d, be):
    mesh = plsc.VectorSubcoreMesh(core_axis_name="c", subcore_axis_name="s")
    epw = be // _NW
    return pl.kernel(
        _sc_msg_body,
        out_type=jax.ShapeDtypeStruct((2 * bn, d), jnp.float32),
        mesh=mesh,
        scratch_types=[
            pltpu.VMEM((epw // 128, 128), jnp.int32),
            pltpu.VMEM((epw // 128, 128), jnp.int32),
            pltpu.VMEM((epw, d), jnp.float32),
            pltpu.VMEM((epw, d), jnp.float32),
            pltpu.VMEM((bn // _NS, d), jnp.float32),
            pltpu.VMEM_SHARED((bn, d), jnp.float32),
            pltpu.SemaphoreType.DMA,
        ],
    )


def _tc_first_body(x_ref, w_ref, h_ref):
    h_ref[...] = jnp.dot(x_ref[...], w_ref[...],
                         preferred_element_type=jnp.float32)


def _tc_mid_body(p_ref, b_ref, g_ref, be_ref, xp_ref, w_ref, x_ref, h_ref,
                 *, first):
    out = p_ref[0] + p_ref[1] + b_ref[...]
    x = jnp.maximum(out, 0.0)
    mu = jnp.mean(x, axis=-1, keepdims=True)
    var = jnp.mean((x - mu) * (x - mu), axis=-1, keepdims=True)
    x = (x - mu) * lax.rsqrt(var + 1e-5) * g_ref[...] + be_ref[...]
    if not first:
        x = x + xp_ref[...]
    x_ref[...] = x
    h_ref[...] = jnp.dot(x, w_ref[...], preferred_element_type=jnp.float32)


def _tc_last_body(p_ref, b_ref, g_ref, be_ref, xp_ref, x0_ref, ge_ref,
                  o_ref):
    out = p_ref[0] + p_ref[1] + b_ref[...]
    x = jnp.maximum(out, 0.0)
    mu = jnp.mean(x, axis=-1, keepdims=True)
    var = jnp.mean((x - mu) * (x - mu), axis=-1, keepdims=True)
    x = (x - mu) * lax.rsqrt(var + 1e-5) * g_ref[...] + be_ref[...]
    x = x + xp_ref[...]
    presence = (jnp.sum(x0_ref[...], axis=-1, keepdims=True) != 0.0
                ).astype(jnp.float32)
    o_ref[...] = x * presence + ge_ref[...] * (1.0 - presence)


def kernel(node_features, edge_indices, edge_attrs, W, b, gamma, beta,
           global_emb):
    bsz, n, d = node_features.shape
    e = edge_attrs.shape[1]
    bn = bsz * n
    be = bsz * e

    ei = edge_indices.astype(jnp.int32)
    goff = (jnp.arange(bsz, dtype=jnp.int32) * n)[:, None]
    src = (ei[:, 0, :] + goff).reshape(be // 128, 128)
    dst = (ei[:, 1, :] + goff).reshape(be // 128, 128)
    ea = edge_attrs.reshape(be, d)
    x0 = node_features.reshape(bn, d)
    ge_t = jnp.tile(global_emb, (bsz, 1))

    sc_msg = _make_sc_msg(bn, d, be)

    f32 = jnp.float32
    mat = jax.ShapeDtypeStruct((bn, d), f32)
    tc_first = pl.pallas_call(_tc_first_body, out_shape=mat)
    tc_last = pl.pallas_call(_tc_last_body, out_shape=mat)

    h = tc_first(x0, W[0])
    x_prev = x0
    for i in range(_L):
        p = sc_msg(h, src, dst, ea).reshape(2, bn, d)
        if i < _L - 1:
            tc_mid = pl.pallas_call(
                functools.partial(_tc_mid_body, first=(i == 0)),
                out_shape=(mat, mat))
            x_new, h = tc_mid(p, b[i:i + 1], gamma[i:i + 1], beta[i:i + 1],
                              x_prev, W[i + 1])
            x_prev = x_new
        else:
            out = tc_last(p, b[i:i + 1], gamma[i:i + 1], beta[i:i + 1],
                          x_prev, x0, ge_t)
    return out.reshape(bsz, n, d)


# speedup vs baseline: 1.7094x; 1.7094x over previous
import jax
import jax.numpy as jnp
from jax import lax
from jax.experimental import pallas as pl
from jax.experimental.pallas import tpu as pltpu
from jax.experimental.pallas import tpu_sc as plsc

_L = 3
_NC, _NS, _LANES = 2, 16, 16


def _sc_adj_body(src_hbm, dst_hbm, ea_hbm, zeros_hbm, a_hbm,
                 src_v, dst_v, cidx_v, ea_v, a_sh, sem_z, sem_e):
    c = lax.axis_index("c")
    s = lax.axis_index("s")
    nsq = a_sh.shape[0] // 2
    epw = ea_v.shape[0]
    nchunk = epw // 128
    zrows = a_sh.shape[0] // _NS
    ebase = c * (_NS * epw) + s * epw

    zcp = pltpu.make_async_copy(zeros_hbm.at[pl.ds(s * zrows, zrows)],
                                a_sh.at[pl.ds(s * zrows, zrows)], sem_z)
    zcp.start()
    ecp = pltpu.make_async_copy(ea_hbm.at[pl.ds(ebase, epw)], ea_v, sem_e)
    ecp.start()

    cw = c * _NS + s
    pltpu.sync_copy(src_hbm.at[cw], src_v)
    pltpu.sync_copy(dst_hbm.at[cw], dst_v)

    off = (s // 8) * nsq
    off_v = jnp.full((_LANES,), off, jnp.int32)
    for ch in range(nchunk):
        for i in range(128 // _LANES):
            sl = pl.ds(i * _LANES, _LANES)
            cidx_v[ch, sl] = (src_v[ch, sl] * 64
                              + dst_v[ch, sl] + off_v)

    ecp.wait()
    zcp.wait()
    plsc.subcore_barrier()

    for ch in range(nchunk):
        pltpu.sync_copy(ea_v.at[pl.ds(ch * 128, 128)],
                        a_sh.at[cidx_v.at[ch]], add=True)
    plsc.subcore_barrier()

    base = c * a_sh.shape[0] + s * zrows
    pltpu.sync_copy(a_sh.at[pl.ds(s * zrows, zrows)],
                    a_hbm.at[pl.ds(base, zrows)])


def _make_sc_adj(bsz, n, d, be):
    mesh = plsc.VectorSubcoreMesh(core_axis_name="c", subcore_axis_name="s")
    epw = be // (_NC * _NS)
    sc_rows = (bsz // _NC) * n * n
    return pl.kernel(
        _sc_adj_body,
        out_type=jax.ShapeDtypeStruct((bsz * n * n, d), jnp.float32),
        mesh=mesh,
        scratch_types=[
            pltpu.VMEM((epw // 128, 128), jnp.int32),
            pltpu.VMEM((epw // 128, 128), jnp.int32),
            pltpu.VMEM((epw // 128, 128), jnp.int32),
            pltpu.VMEM((epw, d), jnp.float32),
            pltpu.VMEM_SHARED((sc_rows, d), jnp.float32),
            pltpu.SemaphoreType.DMA,
            pltpu.SemaphoreType.DMA,
        ],
    )


def _tc_body(x_ref, a_ref, w_ref, b_ref, g_ref, be_ref, ge_ref, out_ref):
    x0 = x_ref[0]
    n = x0.shape[0]
    x = x0
    for i in range(_L):
        residual = x
        h = jnp.dot(x, w_ref[i], preferred_element_type=jnp.float32)
        out = jnp.zeros_like(h)
        for m in range(n):
            out = out + a_ref[0, m] * h[m:m + 1, :]
        out = out + b_ref[i]
        x = jnp.maximum(out, 0.0)
        mu = jnp.mean(x, axis=-1, keepdims=True)
        var = jnp.mean((x - mu) * (x - mu), axis=-1, keepdims=True)
        x = (x - mu) * lax.rsqrt(var + 1e-5) * g_ref[i] + be_ref[i]
        if i > 0:
            x = x + residual

    presence = (jnp.sum(x0, axis=1, keepdims=True) != 0.0
                ).astype(jnp.float32)
    out_ref[0] = x * presence + ge_ref[...] * (1.0 - presence)


def kernel(node_features, edge_indices, edge_attrs, W, b, gamma, beta,
           global_emb):
    bsz, n, d = node_features.shape
    e = edge_attrs.shape[1]
    be = bsz * e

    ei = edge_indices.astype(jnp.int32)
    nw = _NC * _NS
    src = ei[:, 0, :].reshape(nw, be // (nw * 128), 128)
    dst = ei[:, 1, :].reshape(nw, be // (nw * 128), 128)
    ea = edge_attrs.reshape(be, d)
    zeros = jnp.zeros(((bsz // _NC) * n * n, d), jnp.float32)

    a_flat = _make_sc_adj(bsz, n, d, be)(src, dst, ea, zeros)
    a = a_flat.reshape(bsz, n, n, d)

    grid = (bsz,)
    out = pl.pallas_call(
        _tc_body,
        grid=grid,
        in_specs=[
            pl.BlockSpec((1, n, d), lambda g: (g, 0, 0)),
            pl.BlockSpec((1, n, n, d), lambda g: (g, 0, 0, 0)),
            pl.BlockSpec((_L, d, d), lambda g: (0, 0, 0)),
            pl.BlockSpec((_L, d), lambda g: (0, 0)),
            pl.BlockSpec((_L, d), lambda g: (0, 0)),
            pl.BlockSpec((_L, d), lambda g: (0, 0)),
            pl.BlockSpec((n, d), lambda g: (0, 0)),
        ],
        out_specs=pl.BlockSpec((1, n, d), lambda g: (g, 0, 0)),
        out_shape=jax.ShapeDtypeStruct((bsz, n, d), jnp.float32),
    )(node_features, a, W, b, gamma, beta, global_emb)
    return out


# speedup vs baseline: 1.8650x; 1.0910x over previous
import jax
import jax.numpy as jnp
from jax import lax
from jax.experimental import pallas as pl
from jax.experimental.pallas import tpu as pltpu
from jax.experimental.pallas import tpu_sc as plsc

_L = 3
_NC, _NS, _LANES = 2, 16, 16


def _sc_adj_body(src_hbm, dst_hbm, ea_hbm, a_hbm,
                 src_v, dst_v, cidx_v, ea_v, zfill_v, a_sh, sem_z, sem_e):
    c = lax.axis_index("c")
    s = lax.axis_index("s")
    nsq = a_sh.shape[0] // 2
    epw = ea_v.shape[0]
    nchunk = epw // 128
    zrows = a_sh.shape[0] // _NS
    ebase = c * (_NS * epw) + s * epw

    ecp = pltpu.make_async_copy(ea_hbm.at[pl.ds(ebase, epw)], ea_v, sem_e)
    ecp.start()

    zr = zfill_v.shape[0]

    def _zf(i, carry):
        for j in range(8):
            zfill_v[i, pl.ds(j * _LANES, _LANES)] = jnp.zeros(
                (_LANES,), jnp.float32)
        return carry
    lax.fori_loop(0, zr, _zf, 0)
    zcps = [pltpu.make_async_copy(
        zfill_v, a_sh.at[pl.ds(s * zrows + k * zr, zr)], sem_z)
        for k in range(zrows // zr)]
    for cp in zcps:
        cp.start()

    cw = c * _NS + s
    pltpu.sync_copy(src_hbm.at[cw], src_v)
    pltpu.sync_copy(dst_hbm.at[cw], dst_v)

    off = (s // 8) * nsq
    off_v = jnp.full((_LANES,), off, jnp.int32)
    for ch in range(nchunk):
        for i in range(128 // _LANES):
            sl = pl.ds(i * _LANES, _LANES)
            cidx_v[ch, sl] = (src_v[ch, sl] * 64
                              + dst_v[ch, sl] + off_v)

    ecp.wait()
    for cp in zcps:
        cp.wait()
    plsc.subcore_barrier()

    for ch in range(nchunk):
        pltpu.sync_copy(ea_v.at[pl.ds(ch * 128, 128)],
                        a_sh.at[cidx_v.at[ch]], add=True)
    plsc.subcore_barrier()

    base = c * a_sh.shape[0] + s * zrows
    pltpu.sync_copy(a_sh.at[pl.ds(s * zrows, zrows)],
                    a_hbm.at[pl.ds(base, zrows)])


def _make_sc_adj(bsz, n, d, be):
    mesh = plsc.VectorSubcoreMesh(core_axis_name="c", subcore_axis_name="s")
    epw = be // (_NC * _NS)
    sc_rows = (bsz // _NC) * n * n
    return pl.kernel(
        _sc_adj_body,
        out_type=jax.ShapeDtypeStruct((bsz * n * n, d), jnp.float32),
        mesh=mesh,
        scratch_types=[
            pltpu.VMEM((epw // 128, 128), jnp.int32),
            pltpu.VMEM((epw // 128, 128), jnp.int32),
            pltpu.VMEM((epw // 128, 128), jnp.int32),
            pltpu.VMEM((epw, d), jnp.float32),
            pltpu.VMEM((128, d), jnp.float32),
            pltpu.VMEM_SHARED((sc_rows, d), jnp.float32),
            pltpu.SemaphoreType.DMA,
            pltpu.SemaphoreType.DMA,
        ],
    )


def _tc_body(x_ref, a_ref, w_ref, b_ref, g_ref, be_ref, ge_ref, out_ref):
    x0 = x_ref[0]
    n = x0.shape[0]
    x = x0
    for i in range(_L):
        residual = x
        h = jnp.dot(x, w_ref[i], preferred_element_type=jnp.float32)
        out = jnp.zeros_like(h)
        for m in range(n):
            out = out + a_ref[0, m] * h[m:m + 1, :]
        out = out + b_ref[i]
        x = jnp.maximum(out, 0.0)
        mu = jnp.mean(x, axis=-1, keepdims=True)
        var = jnp.mean((x - mu) * (x - mu), axis=-1, keepdims=True)
        x = (x - mu) * lax.rsqrt(var + 1e-5) * g_ref[i] + be_ref[i]
        if i > 0:
            x = x + residual

    presence = (jnp.sum(x0, axis=1, keepdims=True) != 0.0
                ).astype(jnp.float32)
    out_ref[0] = x * presence + ge_ref[...] * (1.0 - presence)


def kernel(node_features, edge_indices, edge_attrs, W, b, gamma, beta,
           global_emb):
    bsz, n, d = node_features.shape
    e = edge_attrs.shape[1]
    be = bsz * e

    ei = edge_indices.astype(jnp.int32)
    nw = _NC * _NS
    src = ei[:, 0, :].reshape(nw, be // (nw * 128), 128)
    dst = ei[:, 1, :].reshape(nw, be // (nw * 128), 128)
    ea = edge_attrs.reshape(be, d)

    a_flat = _make_sc_adj(bsz, n, d, be)(src, dst, ea)
    a = a_flat.reshape(bsz, n, n, d)

    grid = (bsz,)
    out = pl.pallas_call(
        _tc_body,
        grid=grid,
        in_specs=[
            pl.BlockSpec((1, n, d), lambda g: (g, 0, 0)),
            pl.BlockSpec((1, n, n, d), lambda g: (g, 0, 0, 0)),
            pl.BlockSpec((_L, d, d), lambda g: (0, 0, 0)),
            pl.BlockSpec((_L, d), lambda g: (0, 0)),
            pl.BlockSpec((_L, d), lambda g: (0, 0)),
            pl.BlockSpec((_L, d), lambda g: (0, 0)),
            pl.BlockSpec((n, d), lambda g: (0, 0)),
        ],
        out_specs=pl.BlockSpec((1, n, d), lambda g: (g, 0, 0)),
        out_shape=jax.ShapeDtypeStruct((bsz, n, d), jnp.float32),
    )(node_features, a, W, b, gamma, beta, global_emb)
    return out
